# baseline (speedup 1.0000x reference)
import functools

import jax
import jax.numpy as jnp
from jax import lax
from jax.experimental import pallas as pl
from jax.experimental.pallas import tpu as pltpu
from jax.experimental.pallas import tpu_sc as plsc

NCB = 8
CBD = 8
NCODES = 256
D = NCB * CBD
N = 4096 * 50

NW = 32
N_W = N // NW
CH = 800
NCHUNK = N_W // CH
BATCHES = [(k * 128, 128) for k in range(6)] + [(768, 32)]
GGRP = CH // 16

_mesh = plsc.VectorSubcoreMesh(core_axis_name="c", subcore_axis_name="s")

_PBLK = 4096


def _pack_body(r_ref, w0_ref, w1_ref):
    r = r_ref[...]
    w0_ref[...] = r[0] | (r[1] << 8) | (r[2] << 16) | (r[3] << 24)
    w1_ref[...] = r[4] | (r[5] << 8) | (r[6] << 16) | (r[7] << 24)


def _pack_words(rpq):
    nemb = rpq.shape[1]
    return pl.pallas_call(
        _pack_body,
        grid=(nemb // _PBLK,),
        in_specs=[pl.BlockSpec((NCB, _PBLK), lambda i: (0, i))],
        out_specs=[
            pl.BlockSpec((_PBLK,), lambda i: (i,)),
            pl.BlockSpec((_PBLK,), lambda i: (i,)),
        ],
        out_shape=[
            jax.ShapeDtypeStruct((nemb,), jnp.int32),
            jax.ShapeDtypeStruct((nemb,), jnp.int32),
        ],
    )(rpq)


@functools.partial(
    pl.kernel,
    mesh=_mesh,
    compiler_params=pltpu.CompilerParams(needs_layout_passes=False),
    out_type=jax.ShapeDtypeStruct((N * D,), jnp.float32),
    scratch_types=[
        pltpu.VMEM((N_W,), jnp.int32),
        pltpu.VMEM((CH,), jnp.int32),
        pltpu.VMEM((CH,), jnp.int32),
        pltpu.VMEM((CH,), jnp.int32),
        pltpu.VMEM((CH,), jnp.int32),
        pltpu.VMEM((NCB * NCODES * CBD,), jnp.float32),
        pltpu.VMEM((CH * D,), jnp.float32),
        pltpu.VMEM((CH * D,), jnp.float32),
        pltpu.SemaphoreType.DMA,
        pltpu.SemaphoreType.DMA,
        pltpu.SemaphoreType.DMA,
        pltpu.SemaphoreType.DMA,
    ],
)
def _rpq_sc(ids_hbm, w0_hbm, w1_hbm, cb_hbm, out_hbm, ids_v, c0a, c0b,
            c1a, c1b, cb_v, outa, outb, gsem0, gsem1, osem0, osem1):
    wid = lax.axis_index("c") * 16 + lax.axis_index("s")
    base = wid * N_W
    codes0 = (c0a, c0b)
    codes1 = (c1a, c1b)
    out_v = (outa, outb)
    gsems = (gsem0, gsem1)
    osems = (osem0, osem1)

    pltpu.sync_copy(ids_hbm.at[pl.ds(base, N_W)], ids_v)
    pltpu.sync_copy(cb_hbm, cb_v)

    lane = lax.iota(jnp.int32, 16)
    half = lane >> 3
    shift_even = half * 8
    shift_odd = 16 + half * 8
    cbase = [(2 * k + half) * (NCODES * CBD) + (lane & 7) for k in range(4)]

    def fire_gathers(c):
        p = c % 2
        return [
            pltpu.async_copy(
                tbl.at[ids_v.at[pl.ds(c * CH + off, sz)]],
                dst[p].at[pl.ds(off, sz)],
                gsems[p],
            )
            for tbl, dst in ((w0_hbm, codes0), (w1_hbm, codes1))
            for off, sz in BATCHES
        ]

    out_copies = {}
    pending = fire_gathers(0)
    for c in range(NCHUNK):
        p = c % 2
        nxt = fire_gathers(c + 1) if c + 1 < NCHUNK else []
        for cp in pending:
            cp.wait()
        pending = nxt

        if c >= 2:
            out_copies[c - 2].wait()

        def group_body(v, inner):
            cw0 = codes0[p][pl.ds(v * 16, 16)]
            cw1 = codes1[p][pl.ds(v * 16, 16)]
            for j in range(16):
                w0s = jnp.broadcast_to(cw0[j], (16,))
                w1s = jnp.broadcast_to(cw1[j], (16,))
                ob = (v * 16 + j) * D
                for k in range(4):
                    w = w0s if k < 2 else w1s
                    shift = shift_even if k % 2 == 0 else shift_odd
                    code = (w >> shift) & 255
                    val = plsc.load_gather(cb_v, [(code << 3) + cbase[k]])
                    out_v[p][pl.ds(ob + k * 16, 16)] = val
            return inner

        lax.fori_loop(0, GGRP, group_body, 0)

        out_copies[c] = pltpu.async_copy(
            out_v[p], out_hbm.at[pl.ds((base + c * CH) * D, CH * D)],
            osems[p])

    out_copies[NCHUNK - 2].wait()
    out_copies[NCHUNK - 1].wait()


def kernel(input, rpq_indices, codebooks):
    ids = input.reshape(-1)
    w0, w1 = _pack_words(rpq_indices)
    cbf = codebooks.reshape(-1)
    out = _rpq_sc(ids, w0, w1, cbf)
    return out.reshape(input.shape + (D,))


# speedup vs baseline: 1.1197x; 1.1197x over previous
import functools

import jax
import jax.numpy as jnp
from jax import lax
from jax.experimental import pallas as pl
from jax.experimental.pallas import tpu as pltpu
from jax.experimental.pallas import tpu_sc as plsc

NCB = 8
CBD = 8
NCODES = 256
D = NCB * CBD
N = 4096 * 50

NW = 32
N_W = N // NW
CH = 800
NCHUNK = N_W // CH
BATCHES = [(k * 128, 128) for k in range(6)] + [(768, 32)]
GGRP = CH // 16

_mesh = plsc.VectorSubcoreMesh(core_axis_name="c", subcore_axis_name="s")


@functools.partial(
    pl.kernel,
    mesh=_mesh,
    compiler_params=pltpu.CompilerParams(needs_layout_passes=False),
    out_type=jax.ShapeDtypeStruct((N * D,), jnp.float32),
    scratch_types=[
        pltpu.VMEM((N_W,), jnp.int32),
        pltpu.VMEM((CH,), jnp.int32),
        pltpu.VMEM((CH,), jnp.int32),
        pltpu.VMEM((CH,), jnp.int32),
        pltpu.VMEM((CH,), jnp.int32),
        pltpu.VMEM((NCB * NCODES * CBD,), jnp.float32),
        pltpu.VMEM((CH * D,), jnp.float32),
        pltpu.VMEM((CH * D,), jnp.float32),
        pltpu.SemaphoreType.DMA,
        pltpu.SemaphoreType.DMA,
        pltpu.SemaphoreType.DMA,
        pltpu.SemaphoreType.DMA,
    ],
)
def _rpq_sc(ids_hbm, w0_hbm, w1_hbm, cb_hbm, out_hbm, ids_v, c0a, c0b,
            c1a, c1b, cb_v, outa, outb, gsem0, gsem1, osem0, osem1):
    wid = lax.axis_index("c") * 16 + lax.axis_index("s")
    base = wid * N_W
    codes0 = (c0a, c0b)
    codes1 = (c1a, c1b)
    out_v = (outa, outb)
    gsems = (gsem0, gsem1)
    osems = (osem0, osem1)

    pltpu.sync_copy(ids_hbm.at[pl.ds(base, N_W)], ids_v)
    pltpu.sync_copy(cb_hbm, cb_v)

    lane = lax.iota(jnp.int32, 16)
    half = lane >> 3
    shift_even = half * 8
    shift_odd = 16 + half * 8
    cbase = [(2 * k + half) * (NCODES * CBD) + (lane & 7) for k in range(4)]

    def fire_gathers(c):
        p = c % 2
        return [
            pltpu.async_copy(
                tbl.at[ids_v.at[pl.ds(c * CH + off, sz)]],
                dst[p].at[pl.ds(off, sz)],
                gsems[p],
            )
            for tbl, dst in ((w0_hbm, codes0), (w1_hbm, codes1))
            for off, sz in BATCHES
        ]

    out_copies = {}
    pending = fire_gathers(0)
    for c in range(NCHUNK):
        p = c % 2
        nxt = fire_gathers(c + 1) if c + 1 < NCHUNK else []
        for cp in pending:
            cp.wait()
        pending = nxt

        if c >= 2:
            out_copies[c - 2].wait()

        def group_body(v, inner):
            cw0 = codes0[p][pl.ds(v * 16, 16)]
            cw1 = codes1[p][pl.ds(v * 16, 16)]
            for j in range(16):
                w0s = jnp.broadcast_to(cw0[j], (16,))
                w1s = jnp.broadcast_to(cw1[j], (16,))
                ob = (v * 16 + j) * D
                for k in range(4):
                    w = w0s if k < 2 else w1s
                    shift = shift_even if k % 2 == 0 else shift_odd
                    code = (w >> shift) & 255
                    val = plsc.load_gather(cb_v, [(code << 3) + cbase[k]])
                    out_v[p][pl.ds(ob + k * 16, 16)] = val
            return inner

        lax.fori_loop(0, GGRP, group_body, 0)

        out_copies[c] = pltpu.async_copy(
            out_v[p], out_hbm.at[pl.ds((base + c * CH) * D, CH * D)],
            osems[p])

    out_copies[NCHUNK - 2].wait()
    out_copies[NCHUNK - 1].wait()


def kernel(input, rpq_indices, codebooks):
    ids = input.reshape(-1)
    r = rpq_indices
    w0 = r[0] | (r[1] << 8) | (r[2] << 16) | (r[3] << 24)
    w1 = r[4] | (r[5] << 8) | (r[6] << 16) | (r[7] << 24)
    cbf = codebooks.reshape(-1)
    out = _rpq_sc(ids, w0, w1, cbf)
    return out.reshape(input.shape + (D,))


# speedup vs baseline: 1.3689x; 1.2225x over previous
import functools

import jax
import jax.numpy as jnp
from jax import lax
from jax.experimental import pallas as pl
from jax.experimental.pallas import tpu as pltpu
from jax.experimental.pallas import tpu_sc as plsc

NCB = 8
CBD = 8
NCODES = 256
D = NCB * CBD
N = 4096 * 50

NW = 32
N_W = N // NW
CH = 800
NCHUNK = N_W // CH
BATCHES = [(k * 128, 128) for k in range(6)] + [(768, 32)]
GGRP = CH // 16

_mesh = plsc.VectorSubcoreMesh(core_axis_name="c", subcore_axis_name="s")


@functools.partial(
    pl.kernel,
    mesh=_mesh,
    compiler_params=pltpu.CompilerParams(needs_layout_passes=False),
    out_type=jax.ShapeDtypeStruct((N * D,), jnp.float32),
    scratch_types=[
        pltpu.VMEM((N_W,), jnp.int32),
        pltpu.VMEM((CH,), jnp.int32),
        pltpu.VMEM((CH,), jnp.int32),
        pltpu.VMEM((CH,), jnp.int32),
        pltpu.VMEM((CH,), jnp.int32),
        pltpu.VMEM((NCB * NCODES * CBD,), jnp.float32),
        pltpu.VMEM((CH * D,), jnp.float32),
        pltpu.VMEM((CH * D,), jnp.float32),
        pltpu.SemaphoreType.DMA,
        pltpu.SemaphoreType.DMA,
        pltpu.SemaphoreType.DMA,
        pltpu.SemaphoreType.DMA,
    ],
)
def _rpq_sc(ids_hbm, w0_hbm, w1_hbm, cb_hbm, out_hbm, ids_v, c0a, c0b,
            c1a, c1b, cb_v, outa, outb, gsem0, gsem1, osem0, osem1):
    wid = lax.axis_index("c") * 16 + lax.axis_index("s")
    base = wid * N_W
    codes0 = (c0a, c0b)
    codes1 = (c1a, c1b)
    out_v = (outa, outb)
    gsems = (gsem0, gsem1)
    osems = (osem0, osem1)

    pltpu.sync_copy(ids_hbm.at[pl.ds(base, N_W)], ids_v)
    pltpu.sync_copy(cb_hbm, cb_v)

    lane = lax.iota(jnp.int32, 16)
    half = lane >> 3
    shift_even = half * 8
    shift_odd = 16 + half * 8
    cbase = [(2 * k + half) * (NCODES * CBD) + (lane & 7) for k in range(4)]

    def fire_gathers(c):
        p = c % 2
        return [
            pltpu.async_copy(
                tbl.at[ids_v.at[pl.ds(c * CH + off, sz)]],
                dst[p].at[pl.ds(off, sz)],
                gsems[p],
            )
            for tbl, dst in ((w0_hbm, codes0), (w1_hbm, codes1))
            for off, sz in BATCHES
        ]

    out_copies = {}
    pending = fire_gathers(0)
    for c in range(NCHUNK):
        p = c % 2
        nxt = fire_gathers(c + 1) if c + 1 < NCHUNK else []
        for cp in pending:
            cp.wait()
        pending = nxt

        if c >= 2:
            out_copies[c - 2].wait()

        @plsc.parallel_loop(0, GGRP)
        def group_body(v):
            cw0 = codes0[p][pl.ds(v * 16, 16)]
            cw1 = codes1[p][pl.ds(v * 16, 16)]
            for j in range(16):
                w0s = jnp.broadcast_to(cw0[j], (16,))
                w1s = jnp.broadcast_to(cw1[j], (16,))
                ob = (v * 16 + j) * D
                for k in range(4):
                    w = w0s if k < 2 else w1s
                    shift = shift_even if k % 2 == 0 else shift_odd
                    code = (w >> shift) & 255
                    val = plsc.load_gather(cb_v, [(code << 3) + cbase[k]])
                    out_v[p][pl.ds(ob + k * 16, 16)] = val

        out_copies[c] = pltpu.async_copy(
            out_v[p], out_hbm.at[pl.ds((base + c * CH) * D, CH * D)],
            osems[p])

    out_copies[NCHUNK - 2].wait()
    out_copies[NCHUNK - 1].wait()


def kernel(input, rpq_indices, codebooks):
    ids = input.reshape(-1)
    r = rpq_indices
    w0 = r[0] | (r[1] << 8) | (r[2] << 16) | (r[3] << 24)
    w1 = r[4] | (r[5] << 8) | (r[6] << 16) | (r[7] << 24)
    cbf = codebooks.reshape(-1)
    out = _rpq_sc(ids, w0, w1, cbf)
    return out.reshape(input.shape + (D,))


# speedup vs baseline: 1.5267x; 1.1153x over previous
import functools

import jax
import jax.numpy as jnp
from jax import lax
from jax.experimental import pallas as pl
from jax.experimental.pallas import tpu as pltpu
from jax.experimental.pallas import tpu_sc as plsc

NCB = 8
CBD = 8
NCODES = 256
D = NCB * CBD
N = 4096 * 50

NW = 32
N_W = N // NW
CH = 800
NCHUNK = N_W // CH
BATCHES = [(k * 128, 128) for k in range(6)] + [(768, 32)]
GGRP = CH // 16

_mesh = plsc.VectorSubcoreMesh(core_axis_name="c", subcore_axis_name="s")


@functools.partial(
    pl.kernel,
    mesh=_mesh,
    compiler_params=pltpu.CompilerParams(needs_layout_passes=False),
    out_type=jax.ShapeDtypeStruct((N * D,), jnp.float32),
    scratch_types=[
        pltpu.VMEM((N_W,), jnp.int32),
        pltpu.VMEM((CH,), jnp.int32),
        pltpu.VMEM((CH,), jnp.int32),
        pltpu.VMEM((CH,), jnp.int32),
        pltpu.VMEM((CH,), jnp.int32),
        pltpu.VMEM((NCB * NCODES * CBD,), jnp.float32),
        pltpu.VMEM((CH * D,), jnp.float32),
        pltpu.VMEM((CH * D,), jnp.float32),
        pltpu.SemaphoreType.DMA,
        pltpu.SemaphoreType.DMA,
        pltpu.SemaphoreType.DMA,
        pltpu.SemaphoreType.DMA,
    ],
)
def _rpq_sc(ids_hbm, w0_hbm, w1_hbm, cb_hbm, out_hbm, ids_v, c0a, c0b,
            c1a, c1b, cb_v, outa, outb, gsem0, gsem1, osem0, osem1):
    wid = lax.axis_index("c") * 16 + lax.axis_index("s")
    base = wid * N_W
    codes0 = (c0a, c0b)
    codes1 = (c1a, c1b)
    out_v = (outa, outb)
    gsems = (gsem0, gsem1)
    osems = (osem0, osem1)

    pltpu.sync_copy(ids_hbm.at[pl.ds(base, N_W)], ids_v)
    pltpu.sync_copy(cb_hbm, cb_v)

    lane = lax.iota(jnp.int32, 16)
    half = lane >> 3
    shift_even = half * 8
    shift_odd = 16 + half * 8
    cbase = [(2 * k + half) * (NCODES * CBD) + (lane & 7) for k in range(4)]

    def fire_gathers(c):
        p = c % 2
        return [
            pltpu.async_copy(
                tbl.at[ids_v.at[pl.ds(c * CH + off, sz)]],
                dst[p].at[pl.ds(off, sz)],
                gsems[p],
            )
            for tbl, dst in ((w0_hbm, codes0), (w1_hbm, codes1))
            for off, sz in BATCHES
        ]

    out_copies = {}
    pending = fire_gathers(0)
    for c in range(NCHUNK):
        p = c % 2
        nxt = fire_gathers(c + 1) if c + 1 < NCHUNK else []
        for cp in pending:
            cp.wait()
        pending = nxt

        if c >= 2:
            out_copies[c - 2].wait()

        @plsc.parallel_loop(0, GGRP)
        def group_body(v):
            cw0 = codes0[p][pl.ds(v * 16, 16)]
            cw1 = codes1[p][pl.ds(v * 16, 16)]
            for j in range(16):
                w0s = jnp.broadcast_to(cw0[j], (16,))
                w1s = jnp.broadcast_to(cw1[j], (16,))
                ob = (v * 16 + j) * D
                for k in range(4):
                    w = w0s if k < 2 else w1s
                    shift = shift_even if k % 2 == 0 else shift_odd
                    code = (w >> shift) & 255
                    val = plsc.load_gather(cb_v, [(code << 3) + cbase[k]])
                    out_v[p][pl.ds(ob + k * 16, 16)] = val

        out_copies[c] = pltpu.async_copy(
            out_v[p], out_hbm.at[pl.ds((base + c * CH) * D, CH * D)],
            osems[p])

    out_copies[NCHUNK - 2].wait()
    out_copies[NCHUNK - 1].wait()


def kernel(input, rpq_indices, codebooks):
    ids = input.reshape(-1)
    r = rpq_indices
    wts = jnp.array([1, 1 << 8, 1 << 16, 1 << 24], jnp.int32)[:, None]
    w0 = (r[:4] * wts).sum(axis=0)
    w1 = (r[4:] * wts).sum(axis=0)
    cbf = codebooks.reshape(-1)
    out = _rpq_sc(ids, w0, w1, cbf)
    return out.reshape(input.shape + (D,))
